# baseline (speedup 1.0000x reference)
import functools

import jax
import jax.numpy as jnp
from jax import lax
from jax.experimental import pallas as pl
from jax.experimental.pallas import tpu as pltpu
from jax.experimental.pallas import tpu_sc as plsc

ROWS = 2048
W = 32768
NV = W // 16
K = 512
HBITS = 12
HB = 1 << HBITS
SHIFT = 31 - HBITS
NW = 32
RPW = ROWS // NW

_mesh = plsc.VectorSubcoreMesh(core_axis_name="c", subcore_axis_name="s")


def _scalar(v):
    return jnp.reshape(lax.slice(v, (0,), (1,)), ())


@functools.partial(
    pl.kernel,
    out_type=jax.ShapeDtypeStruct((ROWS, W), jnp.float32),
    mesh=_mesh,
    compiler_params=pltpu.CompilerParams(needs_layout_passes=False),
    scratch_types=[
        pltpu.VMEM((W,), jnp.float32),
        pltpu.VMEM((HB + 16,), jnp.int32),
        pltpu.VMEM((W + 16,), jnp.int32),
    ],
)
def _sc_topk(x_hbm, o_hbm, row_v, hist_v, cand_v):
    wid = lax.axis_index("s") * 2 + lax.axis_index("c")
    ones16 = jnp.ones((16,), jnp.int32)
    zero16i = jnp.zeros((16,), jnp.int32)
    zero16f = jnp.zeros((16,), jnp.float32)

    def do_row(rr, carry):
        row = wid * RPW + rr
        pltpu.sync_copy(x_hbm.at[row], row_v)

        @plsc.parallel_loop(0, HB // 16, unroll=8)
        def zb(i):
            hist_v[pl.ds(i * 16, 16)] = zero16i

        @plsc.parallel_loop(0, NV, unroll=8)
        def h1(i):
            v = row_v[pl.ds(i * 16, 16)]
            keys = plsc.bitcast(v, jnp.int32) & jnp.int32(0x7FFFFFFF)
            plsc.addupdate_scatter(hist_v, (keys >> SHIFT,), ones16)

        def hcnt(b):
            return _scalar(hist_v[pl.ds(b, 16)])

        def wcond(bc):
            b, acc = bc
            return acc + hcnt(b) < K

        def wbody(bc):
            b, acc = bc
            return b - 1, acc + hcnt(b)

        bstar, acc = lax.while_loop(wcond, wbody, (jnp.int32(HB - 1), jnp.int32(0)))
        k2 = K - acc

        @plsc.parallel_loop(0, NV, unroll=8, carry=zero16i)
        def cp(i, off):
            v = row_v[pl.ds(i * 16, 16)]
            keys = plsc.bitcast(v, jnp.int32) & jnp.int32(0x7FFFFFFF)
            m = (keys >> SHIFT) == bstar
            mi = jnp.where(m, 1, 0)
            pos = off + plsc.cumsum(mi) - 1
            plsc.store_scatter(cand_v, (pos,), keys, mask=m)
            return off + plsc.all_reduce_population_count(m)
        ncand = _scalar(cp)
        cand_v[pl.ds(ncand, 16)] = zero16i

        nvc = (ncand + 15) >> 4

        def bit_step(j, lo):
            mid = lo | (jnp.int32(1) << (jnp.int32(SHIFT - 1) - j))

            @plsc.parallel_loop(0, nvc, unroll=4, carry=zero16i)
            def av(i, a):
                vk = cand_v[pl.ds(i * 16, 16)]
                return a + jnp.where(vk >= mid, 1, 0)
            return jnp.where(jnp.sum(av) >= k2, mid, lo)
        thr = lax.fori_loop(0, SHIFT, bit_step, bstar << SHIFT)

        @plsc.parallel_loop(0, NV, unroll=8)
        def mp(i):
            v = row_v[pl.ds(i * 16, 16)]
            keys = plsc.bitcast(v, jnp.int32) & jnp.int32(0x7FFFFFFF)
            row_v[pl.ds(i * 16, 16)] = jnp.where(keys >= thr, v, zero16f)

        pltpu.sync_copy(row_v, o_hbm.at[row])
        return carry

    lax.fori_loop(0, RPW, do_row, 0)


def kernel(input):
    x = input
    B, C, _ = x.shape
    out = _sc_topk(x.reshape(ROWS, W))
    return out.reshape(B, C, W)


# speedup vs baseline: 3.4002x; 3.4002x over previous
import functools

import jax
import jax.numpy as jnp
from jax import lax
from jax.experimental import pallas as pl
from jax.experimental.pallas import tpu as pltpu
from jax.experimental.pallas import tpu_sc as plsc

ROWS = 2048
W = 32768
NV = W // 16
K = 512
HBITS = 9
HB = 1 << HBITS
SHIFT = 31 - HBITS
NW = 32
RPW = ROWS // NW

_mesh = plsc.VectorSubcoreMesh(core_axis_name="c", subcore_axis_name="s")


def _scalar(v):
    return jnp.reshape(lax.slice(v, (0,), (1,)), ())


@functools.partial(
    pl.kernel,
    out_type=jax.ShapeDtypeStruct((ROWS * 16,), jnp.float32),
    mesh=_mesh,
    compiler_params=pltpu.CompilerParams(needs_layout_passes=False),
    scratch_types=[
        pltpu.VMEM((W,), jnp.float32),
        pltpu.VMEM((HB * 16,), jnp.int32),
        pltpu.VMEM((W + 16,), jnp.int32),
        pltpu.VMEM((RPW * 16,), jnp.float32),
    ],
)
def _sc_thresh(x_hbm, t_hbm, row_v, hist_v, cand_v, thr_v):
    wid = lax.axis_index("s") * 2 + lax.axis_index("c")
    lane = lax.iota(jnp.int32, 16)
    ones16 = jnp.ones((16,), jnp.int32)
    zero16i = jnp.zeros((16,), jnp.int32)

    def do_row(rr, carry):
        row = wid * RPW + rr
        pltpu.sync_copy(x_hbm.at[row], row_v)

        @plsc.parallel_loop(0, HB, unroll=8)
        def zb(i):
            hist_v[pl.ds(i * 16, 16)] = zero16i

        @plsc.parallel_loop(0, NV, unroll=8, carry=zero16i)
        def h1(i, mk):
            v = row_v[pl.ds(i * 16, 16)]
            keys = plsc.bitcast(v, jnp.int32) & jnp.int32(0x7FFFFFFF)
            plsc.addupdate_scatter(
                hist_v, (((keys >> SHIFT) << 4) + lane,), ones16)
            return jnp.maximum(mk, keys)
        maxk = jnp.max(h1)

        def hcnt(b):
            return jnp.sum(hist_v[pl.ds(b * 16, 16)])

        def wcond(bc):
            b, acc = bc
            return acc + hcnt(b) < K

        def wbody(bc):
            b, acc = bc
            return b - 1, acc + hcnt(b)

        bstar, acc = lax.while_loop(wcond, wbody, (maxk >> SHIFT, jnp.int32(0)))
        k2 = K - acc

        @plsc.parallel_loop(0, NV, unroll=8, carry=zero16i)
        def cp(i, off):
            v = row_v[pl.ds(i * 16, 16)]
            keys = plsc.bitcast(v, jnp.int32) & jnp.int32(0x7FFFFFFF)
            m = (keys >> SHIFT) == bstar
            mi = jnp.where(m, 1, 0)
            pos = off + plsc.cumsum(mi) - 1
            plsc.store_scatter(cand_v, (pos,), keys, mask=m)
            return off + plsc.all_reduce_population_count(m)
        ncand = _scalar(cp)
        cand_v[pl.ds(ncand, 16)] = zero16i

        nvc = (ncand + 15) >> 4

        def bit_step(j, lo):
            mid = lo | (jnp.int32(1) << (jnp.int32(SHIFT - 1) - j))

            @plsc.parallel_loop(0, nvc, unroll=4, carry=zero16i)
            def av(i, a):
                vk = cand_v[pl.ds(i * 16, 16)]
                return a + jnp.where(vk >= mid, 1, 0)
            return jnp.where(jnp.sum(av) >= k2, mid, lo)
        thr = lax.fori_loop(0, SHIFT, bit_step, bstar << SHIFT)

        thr_v[pl.ds(rr * 16, 16)] = plsc.bitcast(zero16i + thr, jnp.float32)
        return carry

    lax.fori_loop(0, RPW, do_row, 0)
    pltpu.sync_copy(thr_v, t_hbm.at[pl.ds(wid * RPW * 16, RPW * 16)])


BR = 32


def _tc_mask(x_ref, t_ref, o_ref):
    x = x_ref[...]
    o_ref[...] = jnp.where(jnp.abs(x) >= t_ref[...], x, jnp.float32(0))


_mask_call = pl.pallas_call(
    _tc_mask,
    grid=(ROWS // BR,),
    in_specs=[
        pl.BlockSpec((BR, W), lambda i: (i, 0)),
        pl.BlockSpec((BR, 1), lambda i: (i, 0)),
    ],
    out_specs=pl.BlockSpec((BR, W), lambda i: (i, 0)),
    out_shape=jax.ShapeDtypeStruct((ROWS, W), jnp.float32),
)


def kernel(input):
    x = input
    B, C, _ = x.shape
    x2 = x.reshape(ROWS, W)
    thr = _sc_thresh(x2).reshape(ROWS, 16)[:, :1]
    out = _mask_call(x2, thr)
    return out.reshape(B, C, W)


# speedup vs baseline: 3.9242x; 1.1541x over previous
import functools

import jax
import jax.numpy as jnp
from jax import lax
from jax.experimental import pallas as pl
from jax.experimental.pallas import tpu as pltpu
from jax.experimental.pallas import tpu_sc as plsc

ROWS = 2048
W = 32768
NV = W // 16
K = 512
HBITS = 9
HB = 1 << HBITS
SHIFT = 31 - HBITS
NW = 32
RPW = ROWS // NW

_mesh = plsc.VectorSubcoreMesh(core_axis_name="c", subcore_axis_name="s")


def _scalar(v):
    return jnp.reshape(lax.slice(v, (0,), (1,)), ())


@functools.partial(
    pl.kernel,
    out_type=jax.ShapeDtypeStruct((ROWS * 16,), jnp.float32),
    mesh=_mesh,
    compiler_params=pltpu.CompilerParams(needs_layout_passes=False),
    scratch_types=[
        pltpu.VMEM((W,), jnp.float32),
        pltpu.VMEM((W,), jnp.float32),
        pltpu.VMEM((HB * 16,), jnp.int32),
        pltpu.VMEM((W + 16,), jnp.int32),
        pltpu.VMEM((RPW * 16,), jnp.float32),
        pltpu.SemaphoreType.DMA,
        pltpu.SemaphoreType.DMA,
    ],
)
def _sc_thresh(x_hbm, t_hbm, row_a, row_b, hist_v, cand_v, thr_v, sem_a, sem_b):
    wid = lax.axis_index("s") * 2 + lax.axis_index("c")
    lane = lax.iota(jnp.int32, 16)
    ones16 = jnp.ones((16,), jnp.int32)
    zero16i = jnp.zeros((16,), jnp.int32)
    base = wid * RPW

    def do_row(rr, row_v):
        @plsc.parallel_loop(0, HB, unroll=8)
        def zb(i):
            hist_v[pl.ds(i * 16, 16)] = zero16i

        @plsc.parallel_loop(0, NV, unroll=8, carry=zero16i)
        def h1(i, mk):
            v = row_v[pl.ds(i * 16, 16)]
            keys = plsc.bitcast(v, jnp.int32) & jnp.int32(0x7FFFFFFF)
            plsc.addupdate_scatter(
                hist_v, (((keys >> SHIFT) << 4) + lane,), ones16)
            return jnp.maximum(mk, keys)
        maxk = jnp.max(h1)

        def hcnt(b):
            return jnp.sum(hist_v[pl.ds(b * 16, 16)])

        def wcond(bc):
            b, acc = bc
            return acc + hcnt(b) < K

        def wbody(bc):
            b, acc = bc
            return b - 1, acc + hcnt(b)

        bstar, acc = lax.while_loop(wcond, wbody, (maxk >> SHIFT, jnp.int32(0)))
        k2 = K - acc

        @plsc.parallel_loop(0, NV, unroll=8, carry=zero16i)
        def cp(i, off):
            v = row_v[pl.ds(i * 16, 16)]
            keys = plsc.bitcast(v, jnp.int32) & jnp.int32(0x7FFFFFFF)
            m = (keys >> SHIFT) == bstar
            mi = jnp.where(m, 1, 0)
            pos = off + plsc.cumsum(mi) - 1
            plsc.store_scatter(cand_v, (pos,), keys, mask=m)
            return off + plsc.all_reduce_population_count(m)
        ncand = _scalar(cp)
        cand_v[pl.ds(ncand, 16)] = zero16i

        nvc = (ncand + 15) >> 4

        def bit_step(j, lo):
            mid = lo | (jnp.int32(1) << (jnp.int32(SHIFT - 1) - j))

            @plsc.parallel_loop(0, nvc, unroll=4, carry=zero16i)
            def av(i, a):
                vk = cand_v[pl.ds(i * 16, 16)]
                return a + jnp.where(vk >= mid, 1, 0)
            return jnp.where(jnp.sum(av) >= k2, mid, lo)
        thr = lax.fori_loop(0, SHIFT, bit_step, bstar << SHIFT)

        thr_v[pl.ds(rr * 16, 16)] = plsc.bitcast(zero16i + thr, jnp.float32)

    pltpu.make_async_copy(x_hbm.at[base], row_a, sem_a).start()

    def do_pair(p, carry):
        r0 = base + 2 * p
        pltpu.make_async_copy(x_hbm.at[r0 + 1], row_b, sem_b).start()
        pltpu.make_async_copy(x_hbm.at[r0], row_a, sem_a).wait()
        do_row(2 * p, row_a)

        @pl.when(p < RPW // 2 - 1)
        def _():
            pltpu.make_async_copy(x_hbm.at[r0 + 2], row_a, sem_a).start()

        pltpu.make_async_copy(x_hbm.at[r0 + 1], row_b, sem_b).wait()
        do_row(2 * p + 1, row_b)
        return carry

    lax.fori_loop(0, RPW // 2, do_pair, 0)
    pltpu.sync_copy(thr_v, t_hbm.at[pl.ds(wid * RPW * 16, RPW * 16)])


BR = 32


def _tc_mask(x_ref, t_ref, o_ref):
    x = x_ref[...]
    o_ref[...] = jnp.where(jnp.abs(x) >= t_ref[...], x, jnp.float32(0))


_mask_call = pl.pallas_call(
    _tc_mask,
    grid=(ROWS // BR,),
    in_specs=[
        pl.BlockSpec((BR, W), lambda i: (i, 0)),
        pl.BlockSpec((BR, 1), lambda i: (i, 0)),
    ],
    out_specs=pl.BlockSpec((BR, W), lambda i: (i, 0)),
    out_shape=jax.ShapeDtypeStruct((ROWS, W), jnp.float32),
)


def kernel(input):
    x = input
    B, C, _ = x.shape
    x2 = x.reshape(ROWS, W)
    thr = _sc_thresh(x2).reshape(ROWS, 16)[:, :1]
    out = _mask_call(x2, thr)
    return out.reshape(B, C, W)
